# baseline (speedup 1.0000x reference)
import functools

import jax
import jax.numpy as jnp
from jax import lax
from jax.experimental import pallas as pl
from jax.experimental.pallas import tpu as pltpu
from jax.experimental.pallas import tpu_sc as plsc

B = 16384
D = 64
L = 16
NC = 2
NS = 16
NW = NC * NS
BPW = B // NW
CH = 128
NCHUNK = BPW // CH
MARGIN = 1.0


def _row_sq_norms(h_rows, r_rows, t_rows, sq_v):
    lanes = lax.iota(jnp.int32, L)

    def lane_sum(v):
        for sh in (8, 4, 2, 1):
            idx = jnp.bitwise_and(lanes + sh, L - 1)
            v = v + v.at[idx].get(mode="promise_in_bounds")
        return v

    def body(g, _):
        vec = jnp.zeros((L,), jnp.float32)
        for j in range(L):
            i = g * L + j
            acc = jnp.zeros((L,), jnp.float32)
            for c in range(D // L):
                sl = pl.ds(c * L, L)
                d = h_rows[i, sl] + r_rows[i, sl] - t_rows[i, sl]
                acc = acc + d * d
            vec = jnp.where(lanes == j, lane_sum(acc), vec)
        sq_v[pl.ds(g * L, L)] = vec
        return 0

    lax.fori_loop(0, BPW // L, body, 0, unroll=False)


def _sc_body(heads2d, rels2d, tails2d, ent_emb, rel_emb, ent_proj, rel_proj,
             pos_out, neg_out,
             h_idx, r_idx, t_idx, h_rows, r_rows, t_rows, sq_v, sem):
    wid = lax.axis_index("s") * NC + lax.axis_index("c")
    base_row = wid * NCHUNK

    pltpu.sync_copy(heads2d.at[pl.ds(base_row, NCHUNK)], h_idx)
    pltpu.sync_copy(rels2d.at[pl.ds(base_row, NCHUNK)], r_idx)
    pltpu.sync_copy(tails2d.at[pl.ds(base_row, NCHUNK)], t_idx)

    for ent_tab, rel_tab, out in ((ent_emb, rel_emb, pos_out),
                                  (ent_proj, rel_proj, neg_out)):
        descs = []
        for j in range(NCHUNK):
            dst = pl.ds(j * CH, CH)
            descs.append(pltpu.async_copy(ent_tab.at[h_idx.at[j]],
                                          h_rows.at[dst], sem))
            descs.append(pltpu.async_copy(rel_tab.at[r_idx.at[j]],
                                          r_rows.at[dst], sem))
            descs.append(pltpu.async_copy(ent_tab.at[t_idx.at[j]],
                                          t_rows.at[dst], sem))
        for desc in descs:
            desc.wait()
        _row_sq_norms(h_rows, r_rows, t_rows, sq_v)
        pltpu.sync_copy(sq_v, out.at[pl.ds(wid * BPW, BPW)])


@functools.cache
def _sc_call():
    mesh = plsc.VectorSubcoreMesh(core_axis_name="c", subcore_axis_name="s",
                                  num_cores=NC, num_subcores=NS)
    return pl.kernel(
        _sc_body,
        out_type=(jax.ShapeDtypeStruct((B,), jnp.float32),
                  jax.ShapeDtypeStruct((B,), jnp.float32)),
        mesh=mesh,
        scratch_types=[
            pltpu.VMEM((NCHUNK, CH), jnp.int32),
            pltpu.VMEM((NCHUNK, CH), jnp.int32),
            pltpu.VMEM((NCHUNK, CH), jnp.int32),
            pltpu.VMEM((BPW, D), jnp.float32),
            pltpu.VMEM((BPW, D), jnp.float32),
            pltpu.VMEM((BPW, D), jnp.float32),
            pltpu.VMEM((BPW,), jnp.float32),
            pltpu.SemaphoreType.DMA,
        ],
        compiler_params=pltpu.CompilerParams(use_tc_tiling_on_sc=False),
    )


def _tc_body(pos_ref, neg_ref, out_ref):
    p = jnp.sqrt(pos_ref[...])
    n = jnp.sqrt(neg_ref[...])
    out_ref[0, 0] = jnp.sum(jnp.maximum(p - n + MARGIN, 0.0)) * (1.0 / B)


_tc_call = pl.pallas_call(
    _tc_body,
    out_shape=jax.ShapeDtypeStruct((1, 1), jnp.float32),
    in_specs=[pl.BlockSpec(memory_space=pltpu.VMEM),
              pl.BlockSpec(memory_space=pltpu.VMEM)],
    out_specs=pl.BlockSpec(memory_space=pltpu.SMEM),
)


def kernel(heads, relations, tails, entity_embedding, relation_embedding,
           entity_projection, relation_projection):
    heads2d = heads.reshape(B // CH, CH)
    rels2d = relations.reshape(B // CH, CH)
    tails2d = tails.reshape(B // CH, CH)
    pos_sq, neg_sq = _sc_call()(heads2d, rels2d, tails2d,
                                entity_embedding, relation_embedding,
                                entity_projection, relation_projection)
    loss = _tc_call(pos_sq.reshape(CH, B // CH), neg_sq.reshape(CH, B // CH))
    return loss[0, 0]


# speedup vs baseline: 1.5235x; 1.5235x over previous
import functools

import jax
import jax.numpy as jnp
from jax import lax
from jax.experimental import pallas as pl
from jax.experimental.pallas import tpu as pltpu
from jax.experimental.pallas import tpu_sc as plsc

B = 16384
D = 64
L = 16
NC = 2
NS = 16
NW = NC * NS
BPW = B // NW
CH = 128
HALF = 256
MARGIN = 1.0


def _row_sq_norms(h_rows, r_rows, t_rows, sq_v, base):
    lanes = lax.iota(jnp.int32, L)

    def lane_sum(v):
        for sh in (8, 4, 2, 1):
            idx = jnp.bitwise_and(lanes + sh, L - 1)
            v = v + v.at[idx].get(mode="promise_in_bounds")
        return v

    def body(g, _):
        vec = jnp.zeros((L,), jnp.float32)
        for j in range(L):
            i = g * L + j
            acc = jnp.zeros((L,), jnp.float32)
            for c in range(D // L):
                sl = pl.ds(c * L, L)
                d = h_rows[i, sl] + r_rows[i, sl] - t_rows[i, sl]
                acc = acc + d * d
            vec = jnp.where(lanes == j, lane_sum(acc), vec)
        sq_v[pl.ds(base + g * L, L)] = vec
        return 0

    lax.fori_loop(0, HALF // L, body, 0, unroll=False)


def _sc_body(heads2d, rels2d, tails2d, ent_emb, rel_emb, ent_proj, rel_proj,
             pos_out, neg_out,
             h_idx, r_idx, t_idx, h_rows, r_rows, t_rows, sq_v, sem):
    wid = lax.axis_index("s") * NC + lax.axis_index("c")
    rows_per_half = HALF // CH

    for ent_tab, rel_tab, out in ((ent_emb, rel_emb, pos_out),
                                  (ent_proj, rel_proj, neg_out)):
        for k in range(BPW // HALF):
            base_row = wid * (BPW // CH) + k * rows_per_half
            sl = pl.ds(base_row, rows_per_half)
            pltpu.sync_copy(heads2d.at[sl], h_idx)
            pltpu.sync_copy(rels2d.at[sl], r_idx)
            pltpu.sync_copy(tails2d.at[sl], t_idx)

            def enq(g, _):
                r0 = g >> 3
                c0 = jnp.bitwise_and(g, (CH // L) - 1) * L
                hv = h_idx[r0, pl.ds(c0, L)]
                rv = r_idx[r0, pl.ds(c0, L)]
                tv = t_idx[r0, pl.ds(c0, L)]
                for j in range(L):
                    i = g * L + j
                    pltpu.async_copy(ent_tab.at[hv[j]], h_rows.at[i], sem)
                    pltpu.async_copy(rel_tab.at[rv[j]], r_rows.at[i], sem)
                    pltpu.async_copy(ent_tab.at[tv[j]], t_rows.at[i], sem)
                return 0

            lax.fori_loop(0, HALF // L, enq, 0, unroll=False)
            pltpu.make_async_copy(ent_emb.at[pl.ds(0, HALF)], h_rows, sem).wait()
            pltpu.make_async_copy(ent_emb.at[pl.ds(0, HALF)], r_rows, sem).wait()
            pltpu.make_async_copy(ent_emb.at[pl.ds(0, HALF)], t_rows, sem).wait()

            _row_sq_norms(h_rows, r_rows, t_rows, sq_v, k * HALF)
        pltpu.sync_copy(sq_v, out.at[pl.ds(wid * BPW, BPW)])


@functools.cache
def _sc_call():
    mesh = plsc.VectorSubcoreMesh(core_axis_name="c", subcore_axis_name="s",
                                  num_cores=NC, num_subcores=NS)
    return pl.kernel(
        _sc_body,
        out_type=(jax.ShapeDtypeStruct((B,), jnp.float32),
                  jax.ShapeDtypeStruct((B,), jnp.float32)),
        mesh=mesh,
        scratch_types=[
            pltpu.VMEM((HALF // CH, CH), jnp.int32),
            pltpu.VMEM((HALF // CH, CH), jnp.int32),
            pltpu.VMEM((HALF // CH, CH), jnp.int32),
            pltpu.VMEM((HALF, D), jnp.float32),
            pltpu.VMEM((HALF, D), jnp.float32),
            pltpu.VMEM((HALF, D), jnp.float32),
            pltpu.VMEM((BPW,), jnp.float32),
            pltpu.SemaphoreType.DMA,
        ],
        compiler_params=pltpu.CompilerParams(use_tc_tiling_on_sc=True),
    )


def _tc_body(pos_ref, neg_ref, out_ref):
    p = jnp.sqrt(pos_ref[...])
    n = jnp.sqrt(neg_ref[...])
    out_ref[0, 0] = jnp.sum(jnp.maximum(p - n + MARGIN, 0.0)) * (1.0 / B)


_tc_call = pl.pallas_call(
    _tc_body,
    out_shape=jax.ShapeDtypeStruct((1, 1), jnp.float32),
    in_specs=[pl.BlockSpec(memory_space=pltpu.VMEM),
              pl.BlockSpec(memory_space=pltpu.VMEM)],
    out_specs=pl.BlockSpec(memory_space=pltpu.SMEM),
)


def kernel(heads, relations, tails, entity_embedding, relation_embedding,
           entity_projection, relation_projection):
    heads2d = heads.reshape(B // CH, CH)
    rels2d = relations.reshape(B // CH, CH)
    tails2d = tails.reshape(B // CH, CH)
    pos_sq, neg_sq = _sc_call()(heads2d, rels2d, tails2d,
                                entity_embedding, relation_embedding,
                                entity_projection, relation_projection)
    loss = _tc_call(pos_sq.reshape(CH, B // CH), neg_sq.reshape(CH, B // CH))
    return loss[0, 0]
